# Initial kernel scaffold; baseline (speedup 1.0000x reference)
import jax
import jax.numpy as jnp
from jax.experimental import pallas as pl


def kernel(inputs, embeddings):
    raise NotImplementedError("write your pallas kernel here")


# speedup vs baseline: 3.9518x; 3.9518x over previous
import functools

import jax
import jax.numpy as jnp
from jax import lax
from jax.experimental import pallas as pl
from jax.experimental.pallas import tpu as pltpu
from jax.experimental.pallas import tpu_sc as plsc

VOCAB = 100000
DIM = 64
BATCH = 4096
HIST = 200
TOTAL = BATCH * HIST

NUM_CORES = 2
NUM_SUBCORES = 16
NUM_WORKERS = NUM_CORES * NUM_SUBCORES
PER_WORKER = TOTAL // NUM_WORKERS

IDX_BLK = 512
GATHER = 128
GATHERS_PER_BLK = IDX_BLK // GATHER
N_BLKS = PER_WORKER // IDX_BLK


def _emb_body(table_hbm, idx_hbm, out_hbm, idx_v, rows_v, sem):
    wid = lax.axis_index("s") * NUM_CORES + lax.axis_index("c")
    base = wid * PER_WORKER

    def blk(i, carry):
        off = base + i * IDX_BLK
        pltpu.sync_copy(idx_hbm.at[pl.ds(off, IDX_BLK)], idx_v)
        copies = []
        for j in range(GATHERS_PER_BLK):
            s = j * GATHER
            copies.append(
                pltpu.async_copy(
                    table_hbm.at[idx_v.at[pl.ds(s, GATHER)]],
                    rows_v.at[pl.ds(s, GATHER)],
                    sem,
                )
            )
        for c in copies:
            c.wait()
        pltpu.sync_copy(rows_v, out_hbm.at[pl.ds(off, IDX_BLK)])
        return carry

    lax.fori_loop(0, N_BLKS, blk, 0)


@jax.jit
def _embed(embeddings, idx_flat):
    mesh = plsc.VectorSubcoreMesh(core_axis_name="c", subcore_axis_name="s")
    f = pl.kernel(
        _emb_body,
        mesh=mesh,
        compiler_params=pltpu.CompilerParams(use_tc_tiling_on_sc=False),
        out_type=jax.ShapeDtypeStruct((TOTAL, DIM), jnp.float32),
        scratch_types=[
            pltpu.VMEM((IDX_BLK,), jnp.int32),
            pltpu.VMEM((IDX_BLK, DIM), jnp.float32),
            pltpu.SemaphoreType.DMA,
        ],
    )
    return f(embeddings, idx_flat)


def kernel(inputs, embeddings):
    idx_flat = inputs.reshape(TOTAL)
    out = _embed(embeddings, idx_flat)
    return out.reshape(BATCH, HIST, DIM)


# speedup vs baseline: 4.2627x; 1.0787x over previous
import jax
import jax.numpy as jnp
from jax import lax
from jax.experimental import pallas as pl
from jax.experimental.pallas import tpu as pltpu
from jax.experimental.pallas import tpu_sc as plsc

VOCAB = 100000
DIM = 64
BATCH = 4096
HIST = 200
TOTAL = BATCH * HIST

NUM_CORES = 2
NUM_SUBCORES = 16
NUM_WORKERS = NUM_CORES * NUM_SUBCORES
PER_WORKER = TOTAL // NUM_WORKERS

IDX_BLK = 640
GATHER = 128
GATHERS_PER_BLK = IDX_BLK // GATHER
N_BLKS = PER_WORKER // IDX_BLK
N_OUTER = N_BLKS // 2


def _emb_body(table_hbm, idx_hbm, out_hbm,
              idx0, idx1, rows0, rows1,
              sem_i0, sem_i1, sem_g0, sem_g1, sem_s0, sem_s1):
    wid = lax.axis_index("s") * NUM_CORES + lax.axis_index("c")
    base = wid * PER_WORKER
    idx_v = (idx0, idx1)
    rows_v = (rows0, rows1)
    sem_i = (sem_i0, sem_i1)
    sem_g = (sem_g0, sem_g1)
    sem_s = (sem_s0, sem_s1)

    for b in range(2):
        pltpu.async_copy(idx_hbm.at[pl.ds(base + b * IDX_BLK, IDX_BLK)],
                         idx_v[b], sem_i[b])

    def outer(o, carry):
        for b in range(2):
            i = o * 2 + b
            off = base + i * IDX_BLK
            pltpu.make_async_copy(idx_hbm.at[pl.ds(base, IDX_BLK)],
                                  idx_v[b], sem_i[b]).wait()

            @pl.when(o >= 1)
            def _():
                pltpu.make_async_copy(rows_v[b],
                                      out_hbm.at[pl.ds(base, IDX_BLK)],
                                      sem_s[b]).wait()

            copies = []
            for j in range(GATHERS_PER_BLK):
                s = j * GATHER
                copies.append(pltpu.async_copy(
                    table_hbm.at[idx_v[b].at[pl.ds(s, GATHER)]],
                    rows_v[b].at[pl.ds(s, GATHER)],
                    sem_g[b]))

            for c in copies:
                c.wait()

            @pl.when(o < N_OUTER - 1)
            def _():
                pltpu.async_copy(idx_hbm.at[pl.ds(off + 2 * IDX_BLK, IDX_BLK)],
                                 idx_v[b], sem_i[b])

            pltpu.async_copy(rows_v[b], out_hbm.at[pl.ds(off, IDX_BLK)],
                             sem_s[b])
        return carry

    lax.fori_loop(0, N_OUTER, outer, 0)

    for b in range(2):
        pltpu.make_async_copy(rows_v[b], out_hbm.at[pl.ds(base, IDX_BLK)],
                              sem_s[b]).wait()


@jax.jit
def _embed(embeddings, idx_flat):
    mesh = plsc.VectorSubcoreMesh(core_axis_name="c", subcore_axis_name="s")
    f = pl.kernel(
        _emb_body,
        mesh=mesh,
        compiler_params=pltpu.CompilerParams(use_tc_tiling_on_sc=False),
        out_type=jax.ShapeDtypeStruct((TOTAL, DIM), jnp.float32),
        scratch_types=[
            pltpu.VMEM((IDX_BLK,), jnp.int32),
            pltpu.VMEM((IDX_BLK,), jnp.int32),
            pltpu.VMEM((IDX_BLK, DIM), jnp.float32),
            pltpu.VMEM((IDX_BLK, DIM), jnp.float32),
            pltpu.SemaphoreType.DMA,
            pltpu.SemaphoreType.DMA,
            pltpu.SemaphoreType.DMA,
            pltpu.SemaphoreType.DMA,
            pltpu.SemaphoreType.DMA,
            pltpu.SemaphoreType.DMA,
        ],
    )
    return f(embeddings, idx_flat)


def kernel(inputs, embeddings):
    idx_flat = inputs.reshape(TOTAL)
    out = _embed(embeddings, idx_flat)
    return out.reshape(BATCH, HIST, DIM)
